# Initial kernel scaffold; baseline (speedup 1.0000x reference)
import jax
import jax.numpy as jnp
from jax import lax
from jax.experimental import pallas as pl
from jax.experimental.pallas import tpu as pltpu
from jax.experimental.pallas import tpu_sc as plsc

VOCAB = 100000
D = 64
N_PAIRS = 4096 * 50
NC = 2
NS = 16
NW = NC * NS
PER_W = N_PAIRS // NW
CHUNK = 128
N_CHUNKS = PER_W // CHUNK
LANES = 16
GROUPS = CHUNK // LANES

_EPS = 1e-12


def _rsqrt_fast(x):
    i = plsc.bitcast(x, jnp.int32)
    i = jnp.int32(0x5F3759DF) - (i >> 1)
    y = plsc.bitcast(i, jnp.float32)
    for _ in range(3):
        y = y * (1.5 - 0.5 * x * y * y)
    return y


def _make_sc_kernel():
    mesh = plsc.VectorSubcoreMesh(core_axis_name="c", subcore_axis_name="s",
                                  num_cores=NC, num_subcores=NS)

    def body(idx_w_hbm, idx_r_hbm, w_word_hbm, w_rel_hbm, out_hbm,
             idx_w_v, idx_r_v, c_rows, t_rows, out_v, sem_c, sem_t):
        wid = lax.axis_index("s") * NC + lax.axis_index("c")
        pltpu.sync_copy(idx_w_hbm.at[wid], idx_w_v)
        pltpu.sync_copy(idx_r_hbm.at[wid], idx_r_v)

        lane_iota = lax.iota(jnp.int32, LANES)

        def compute_chunk(chunk, crows, trows):
            out_base = chunk * CHUNK

            def group_body(g, _):
                riv = g * LANES + lane_iota

                def d_body(d, carry):
                    acc, cc, tt = carry
                    dsp = jnp.zeros((LANES,), jnp.int32) + d
                    c = plsc.load_gather(crows, [riv, dsp])
                    t = plsc.load_gather(trows, [riv, dsp])
                    return acc + c * t, cc + c * c, tt + t * t

                zeros = jnp.zeros((LANES,), jnp.float32)
                acc, cc, tt = lax.fori_loop(0, D, d_body,
                                            (zeros, zeros, zeros))
                denom = jnp.maximum(cc, _EPS) * jnp.maximum(tt, _EPS)
                z = acc * _rsqrt_fast(denom)
                p = 1.0 / (1.0 + jnp.exp(-z))
                out_v[pl.ds(out_base + g * LANES, LANES)] = p
                return 0

            lax.fori_loop(0, GROUPS, group_body, 0)

        def chunk_body(j, _):
            pltpu.async_copy(w_word_hbm.at[idx_w_v.at[j]], c_rows, sem_c).wait()
            pltpu.async_copy(w_rel_hbm.at[idx_r_v.at[j]], t_rows, sem_t).wait()
            compute_chunk(j, c_rows, t_rows)
            return 0

        lax.fori_loop(0, N_CHUNKS, chunk_body, 0)
        pltpu.sync_copy(out_v, out_hbm.at[pl.ds(wid * PER_W, PER_W)])

    return pl.kernel(
        body,
        out_type=jax.ShapeDtypeStruct((N_PAIRS,), jnp.float32),
        mesh=mesh,
        scratch_types=[
            pltpu.VMEM((N_CHUNKS, CHUNK), jnp.int32),
            pltpu.VMEM((N_CHUNKS, CHUNK), jnp.int32),
            pltpu.VMEM((CHUNK, D), jnp.float32),
            pltpu.VMEM((CHUNK, D), jnp.float32),
            pltpu.VMEM((PER_W,), jnp.float32),
            pltpu.SemaphoreType.DMA,
            pltpu.SemaphoreType.DMA,
        ],
    )


_sc_kernel = _make_sc_kernel()


@jax.jit
def kernel(x, W_word, W_rel):
    idx = x.astype(jnp.int32).reshape(N_PAIRS, 2)
    idx_w = idx[:, 0].reshape(NW, N_CHUNKS, CHUNK)
    idx_r = idx[:, 1].reshape(NW, N_CHUNKS, CHUNK)
    out = _sc_kernel(idx_w, idx_r, W_word, W_rel)
    return out.reshape(4096, 50, 1)


# speedup vs baseline: 3.3488x; 3.3488x over previous
import jax
import jax.numpy as jnp
from jax import lax
from jax.experimental import pallas as pl
from jax.experimental.pallas import tpu as pltpu
from jax.experimental.pallas import tpu_sc as plsc

VOCAB = 100000
D = 64
N_PAIRS = 4096 * 50
NC = 2
NS = 16
NW = NC * NS
PER_W = N_PAIRS // NW
CHUNK = 128
N_CHUNKS = PER_W // CHUNK
LANES = 16
GROUPS = CHUNK // LANES

_EPS = 1e-12


def _rsqrt_fast(x):
    i = plsc.bitcast(x, jnp.int32)
    i = jnp.int32(0x5F3759DF) - (i >> 1)
    y = plsc.bitcast(i, jnp.float32)
    for _ in range(3):
        y = y * (1.5 - 0.5 * x * y * y)
    return y


def _make_sc_kernel():
    mesh = plsc.VectorSubcoreMesh(core_axis_name="c", subcore_axis_name="s",
                                  num_cores=NC, num_subcores=NS)

    def body(idx_w_hbm, idx_r_hbm, w_word_hbm, w_rel_hbm, out_hbm,
             idx_w_v, idx_r_v, c_rows, t_rows, out_v, sem_c, sem_t):
        wid = lax.axis_index("s") * NC + lax.axis_index("c")
        pltpu.sync_copy(idx_w_hbm.at[wid], idx_w_v)
        pltpu.sync_copy(idx_r_hbm.at[wid], idx_r_v)

        lane_iota = lax.iota(jnp.int32, LANES)

        def compute_chunk(chunk, crows, trows):
            out_base = chunk * CHUNK

            def group_body(g, _):
                riv = g * LANES + lane_iota

                def d_body(d, carry):
                    acc, cc, tt = carry
                    dsp = jnp.zeros((LANES,), jnp.int32) + d
                    c = plsc.load_gather(crows, [riv, dsp])
                    t = plsc.load_gather(trows, [riv, dsp])
                    return acc + c * t, cc + c * c, tt + t * t

                zeros = jnp.zeros((LANES,), jnp.float32)
                acc, cc, tt = lax.fori_loop(0, D, d_body,
                                            (zeros, zeros, zeros))
                denom = jnp.maximum(cc, _EPS) * jnp.maximum(tt, _EPS)
                z = acc * _rsqrt_fast(denom)
                p = 1.0 / (1.0 + jnp.exp(-z))
                out_v[pl.ds(out_base + g * LANES, LANES)] = p
                return 0

            lax.fori_loop(0, GROUPS, group_body, 0)

        def chunk_body(j, _):
            pltpu.async_copy(w_word_hbm.at[idx_w_v.at[j]], c_rows, sem_c).wait()
            pltpu.async_copy(w_rel_hbm.at[idx_r_v.at[j]], t_rows, sem_t).wait()
            compute_chunk(j, c_rows, t_rows)
            return 0

        lax.fori_loop(0, N_CHUNKS, chunk_body, 0)
        pltpu.sync_copy(out_v, out_hbm.at[pl.ds(wid * PER_W, PER_W)])

    return pl.kernel(
        body,
        out_type=jax.ShapeDtypeStruct((N_PAIRS,), jnp.float32),
        mesh=mesh,
        compiler_params=pltpu.CompilerParams(needs_layout_passes=False,
                                             use_tc_tiling_on_sc=False),
        scratch_types=[
            pltpu.VMEM((N_CHUNKS, CHUNK), jnp.int32),
            pltpu.VMEM((N_CHUNKS, CHUNK), jnp.int32),
            pltpu.VMEM((CHUNK, D), jnp.float32),
            pltpu.VMEM((CHUNK, D), jnp.float32),
            pltpu.VMEM((PER_W,), jnp.float32),
            pltpu.SemaphoreType.DMA,
            pltpu.SemaphoreType.DMA,
        ],
    )


_sc_kernel = _make_sc_kernel()


@jax.jit
def kernel(x, W_word, W_rel):
    idx = x.astype(jnp.int32).reshape(N_PAIRS, 2)
    idx_w = idx[:, 0].reshape(NW, N_CHUNKS, CHUNK)
    idx_r = idx[:, 1].reshape(NW, N_CHUNKS, CHUNK)
    out = _sc_kernel(idx_w, idx_r, W_word, W_rel)
    return out.reshape(4096, 50, 1)
